# baseline (speedup 1.0000x reference)
import functools

import jax
import jax.numpy as jnp
from jax import lax
from jax.experimental import pallas as pl
from jax.experimental.pallas import tpu as pltpu
from jax.experimental.pallas import tpu_sc as plsc

B, I, H, O = 128, 512, 1024, 256
E1, E2 = 65536, 65536
NS = 16
CH = 128
EROWS = E1 // CH
ROWS = EROWS // NS
N1 = I * H
N2 = H * O
PT = N1 // NS
ZCH = 4096
LANES = 16


@functools.cache
def _get_densify():
  mesh = plsc.VectorSubcoreMesh(core_axis_name="c", subcore_axis_name="s",
                                num_cores=2, num_subcores=NS)

  @functools.partial(
      pl.kernel,
      out_type=jax.ShapeDtypeStruct((2 * N1,), jnp.float32),
      mesh=mesh,
      scratch_types=(
          pltpu.VMEM((ROWS, CH), jnp.int32),
          pltpu.VMEM((ROWS, CH), jnp.int32),
          pltpu.VMEM((ROWS, CH), jnp.float32),
          pltpu.VMEM((ROWS, CH), jnp.int32),
          pltpu.VMEM((ZCH,), jnp.float32),
          pltpu.VMEM_SHARED((N1,), jnp.float32),
          pltpu.SemaphoreType.DMA,
          pltpu.SemaphoreType.DMA,
          pltpu.SemaphoreType.DMA,
      ),
  )
  def _densify(ein, eout, wgt, out,
               vin, vout, vw, vidx, vzero, acc, sem_in, sem_z, sem_s):
    cid = lax.axis_index("c")
    sid = lax.axis_index("s")

    rows0 = cid * EROWS + sid * ROWS
    stage = [
        pltpu.async_copy(ein.at[pl.ds(rows0, ROWS)], vin, sem_in),
        pltpu.async_copy(eout.at[pl.ds(rows0, ROWS)], vout, sem_in),
        pltpu.async_copy(wgt.at[pl.ds(rows0, ROWS)], vw, sem_in),
    ]
    zbase = sid * PT

    def vzbody(j, _):
        vzero[pl.ds(j * LANES, LANES)] = jnp.zeros((LANES,), jnp.float32)
        return 0
    lax.fori_loop(0, ZCH // LANES, vzbody, 0)

    def zfill(k, _):
        pltpu.async_copy(vzero, acc.at[pl.ds(zbase + k * ZCH, ZCH)], sem_z)
        return 0
    lax.fori_loop(0, PT // ZCH, zfill, 0)
    for cp in stage:
        cp.wait()

    blkstride = jnp.where(cid == 0, I * CH, H * CH)

    def ibody(r, _):
        for l in range(CH // LANES):
            s = pl.ds(l * LANES, LANES)
            vo = vout[r, s]
            vidx[r, s] = ((vo >> 7) * blkstride + (vin[r, s] << 7)
                          + (vo & (CH - 1)))
        return 0
    lax.fori_loop(0, ROWS, ibody, 0)

    obase = cid * N1 + zbase
    pltpu.make_async_copy(out.at[pl.ds(obase, PT)],
                          acc.at[pl.ds(zbase, PT)], sem_z).wait()
    plsc.subcore_barrier()

    def sbody(r, _):
        pltpu.async_copy(vw.at[r], acc.at[vidx.at[r]], sem_s, add=True)
        return 0
    lax.fori_loop(0, ROWS, sbody, 0)
    pltpu.make_async_copy(wgt.at[pl.ds(0, ROWS)], vw, sem_s).wait()

    plsc.subcore_barrier()

    pltpu.sync_copy(acc.at[pl.ds(zbase, PT)], out.at[pl.ds(obase, PT)])

  return _densify


def _split(a):
    hi = a.astype(jnp.bfloat16)
    lo = (a - hi.astype(jnp.float32)).astype(jnp.bfloat16)
    return hi, lo


def _dot3(a_hi, a_lo, b):
    b_hi, b_lo = _split(b)
    return (jnp.dot(a_hi, b_hi, preferred_element_type=jnp.float32)
            + jnp.dot(a_lo, b_hi, preferred_element_type=jnp.float32)
            + jnp.dot(a_hi, b_lo, preferred_element_type=jnp.float32))


def _mm_body(x_ref, w_ref, o_ref, h_ref):
    x_hi, x_lo = _split(x_ref[...])
    for p in range(H // 256):
        rhs = jnp.concatenate(
            [w_ref[2 * p * I:(2 * p + 1) * I, :],
             w_ref[(2 * p + 1) * I:(2 * p + 2) * I, :]], axis=1)
        h_ref[:, p * 256:(p + 1) * 256] = jnp.maximum(
            _dot3(x_hi, x_lo, rhs), 0.0)
    h_hi, h_lo = _split(h_ref[...])
    w2base = N1 // CH
    rhs2 = jnp.concatenate(
        [w_ref[w2base:w2base + H, :],
         w_ref[w2base + H:w2base + 2 * H, :]], axis=1)
    o_ref[...] = _dot3(h_hi, h_lo, rhs2)


_mm = pl.pallas_call(
    _mm_body,
    out_shape=jax.ShapeDtypeStruct((B, O), jnp.float32),
    scratch_shapes=[pltpu.VMEM((B, H), jnp.float32)],
)


def kernel(x, e1_in, e1_out, w1, e2_in, e2_out, w2):
    ein = jnp.concatenate([e1_in, e2_in]).reshape(2 * EROWS, CH)
    eout = jnp.concatenate([e1_out, e2_out]).reshape(2 * EROWS, CH)
    wgt = jnp.concatenate([w1, w2]).reshape(2 * EROWS, CH)
    wall = _get_densify()(ein, eout, wgt)
    return _mm(x, wall.reshape(2 * N1 // CH, CH))


# speedup vs baseline: 1.0365x; 1.0365x over previous
import functools

import jax
import jax.numpy as jnp
from jax import lax
from jax.experimental import pallas as pl
from jax.experimental.pallas import tpu as pltpu
from jax.experimental.pallas import tpu_sc as plsc

B, I, H, O = 128, 512, 1024, 256
E1, E2 = 65536, 65536
NS = 16
CH = 128
ROWS = (E1 // NS) // CH
N1 = I * H
N2 = H * O
ZCH = 4096
LANES = 16

@functools.cache
def _get_densify():
  mesh = plsc.VectorSubcoreMesh(core_axis_name="c", subcore_axis_name="s",
                                num_cores=2, num_subcores=NS)

  @functools.partial(
      pl.kernel,
      out_type=(
          jax.ShapeDtypeStruct((N1,), jnp.float32),
          jax.ShapeDtypeStruct((N2,), jnp.float32),
      ),
      mesh=mesh,
      scratch_types=(
          pltpu.VMEM((ROWS, CH), jnp.int32),
          pltpu.VMEM((ROWS, CH), jnp.int32),
          pltpu.VMEM((ROWS, CH), jnp.float32),
          pltpu.VMEM((ROWS, CH), jnp.int32),
          pltpu.VMEM((ZCH,), jnp.float32),
          pltpu.VMEM_SHARED((N1,), jnp.float32),
          pltpu.SemaphoreType.DMA,
          pltpu.SemaphoreType.DMA,
          pltpu.SemaphoreType.DMA,
      ),
  )
  def _densify(e1i, e1o, w1e, e2i, e2o, w2e, w1d, w2d,
               vin, vout, vw, vidx, vzero, acc, sem_in, sem_z, sem_s):
    cid = lax.axis_index("c")
    sid = lax.axis_index("s")

    def zero_vzero():
        def body(i, _):
            vzero[pl.ds(i * LANES, LANES)] = jnp.zeros((LANES,), jnp.float32)
            return 0
        lax.fori_loop(0, ZCH // LANES, body, 0)

    def run_layer(ei, eo, we, out, n_dst, nrows):
        rows0 = sid * ROWS
        stage = [
            pltpu.async_copy(ei.at[pl.ds(rows0, ROWS)], vin, sem_in),
            pltpu.async_copy(eo.at[pl.ds(rows0, ROWS)], vout, sem_in),
            pltpu.async_copy(we.at[pl.ds(rows0, ROWS)], vw, sem_in),
        ]
        per_tile = n_dst // NS
        zbase = sid * per_tile

        def zbody(k, _):
            pltpu.async_copy(vzero, acc.at[pl.ds(zbase + k * ZCH, ZCH)], sem_z)
            return 0
        lax.fori_loop(0, per_tile // ZCH, zbody, 0)
        for cp in stage:
            cp.wait()

        blk = nrows * CH

        def ibody(r, _):
            for l in range(CH // LANES):
                s = pl.ds(l * LANES, LANES)
                vo = vout[r, s]
                vidx[r, s] = ((vo >> 7) * blk + (vin[r, s] << 7)
                              + (vo & (CH - 1)))
            return 0
        lax.fori_loop(0, ROWS, ibody, 0)

        pltpu.make_async_copy(
            out.at[pl.ds(zbase, per_tile)],
            acc.at[pl.ds(zbase, per_tile)], sem_z).wait()
        plsc.subcore_barrier()

        def sbody(r, _):
            pltpu.async_copy(vw.at[r], acc.at[vidx.at[r]], sem_s, add=True)
            return 0
        lax.fori_loop(0, ROWS, sbody, 0)
        pltpu.make_async_copy(we.at[pl.ds(0, ROWS)], vw, sem_s).wait()

        plsc.subcore_barrier()

        pltpu.sync_copy(acc.at[pl.ds(zbase, per_tile)],
                        out.at[pl.ds(zbase, per_tile)])

    zero_vzero()

    @pl.when(cid == 0)
    def _():
        run_layer(e1i, e1o, w1e, w1d, N1, I)

    @pl.when(cid == 1)
    def _():
        run_layer(e2i, e2o, w2e, w2d, N2, H)

  return _densify


def _split(a):
    hi = a.astype(jnp.bfloat16)
    lo = (a - hi.astype(jnp.float32)).astype(jnp.bfloat16)
    return hi, lo


def _dot3(a_hi, a_lo, b):
    b_hi, b_lo = _split(b)
    return (jnp.dot(a_hi, b_hi, preferred_element_type=jnp.float32)
            + jnp.dot(a_lo, b_hi, preferred_element_type=jnp.float32)
            + jnp.dot(a_hi, b_lo, preferred_element_type=jnp.float32))


def _mm_body(x_ref, w1_ref, w2_ref, o_ref, h_ref):
    x_hi, x_lo = _split(x_ref[...])
    for p in range(H // 256):
        rhs = jnp.concatenate([w1_ref[2 * p], w1_ref[2 * p + 1]], axis=1)
        h_ref[:, p * 256:(p + 1) * 256] = jnp.maximum(
            _dot3(x_hi, x_lo, rhs), 0.0)
    h_hi, h_lo = _split(h_ref[...])
    o_ref[...] = _dot3(h_hi, h_lo,
                       jnp.concatenate([w2_ref[0], w2_ref[1]], axis=1))


_mm = pl.pallas_call(
    _mm_body,
    out_shape=jax.ShapeDtypeStruct((B, O), jnp.float32),
    scratch_shapes=[pltpu.VMEM((B, H), jnp.float32)],
)


def kernel(x, e1_in, e1_out, w1, e2_in, e2_out, w2):
    w1d, w2d = _get_densify()(
        e1_in.reshape(E1 // CH, CH), e1_out.reshape(E1 // CH, CH),
        w1.reshape(E1 // CH, CH),
        e2_in.reshape(E2 // CH, CH), e2_out.reshape(E2 // CH, CH),
        w2.reshape(E2 // CH, CH),
    )
    return _mm(x, w1d.reshape(H // 128, I, 128),
               w2d.reshape(O // 128, H, 128))
